# baseline (speedup 1.0000x reference)
import functools

import jax
import jax.numpy as jnp
from jax import lax
from jax.experimental import pallas as pl
from jax.experimental.pallas import tpu as pltpu
from jax.experimental.pallas import tpu_sc as plsc

NC = 2
NS = 16
L = 16
NW = NC * NS

N_ROWS = 16384
WIDTH = 1920
N_TOTAL = N_ROWS * WIDTH
PER_W = N_TOTAL // NW
T = 24576
NCH = PER_W // T
UNROLL = 8

assert PER_W * NW == N_TOTAL and NCH * T == PER_W and T % (L * UNROLL) == 0

_TBL = 48


def _sc_body(x_hbm, g_hbm, out_hbm, xbuf0, xbuf1, ybuf0, ybuf1, gt, dt,
             sin0, sin1, sout0, sout1):
    wid = lax.axis_index("c") * NS + lax.axis_index("s")
    base = wid * PER_W
    xbufs = (xbuf0, xbuf1)
    ybufs = (ybuf0, ybuf1)
    sins = (sin0, sin1)
    souts = (sout0, sout1)

    pltpu.sync_copy(g_hbm, gt)
    for k in range(_TBL // L):
        i0 = lax.iota(jnp.int32, L) + (16 * k)
        i0 = jnp.minimum(i0, 32)
        i1 = jnp.minimum(i0 + 1, 32)
        dt[pl.ds(16 * k, L)] = (plsc.load_gather(gt, [i1])
                                - plsc.load_gather(gt, [i0]))

    def in_copy(c, b):
        return pltpu.make_async_copy(
            x_hbm.at[pl.ds(base + c * T, T)], xbufs[b], sins[b])

    def out_copy(c, b):
        return pltpu.make_async_copy(
            ybufs[b], out_hbm.at[pl.ds(base + c * T, T)], souts[b])

    in_copy(0, 0).start()
    in_copy(1, 1).start()

    def compute_chunk(b):
        xb = xbufs[b]
        yb = ybufs[b]

        def vec(i, carry):
            for u in range(UNROLL):
                o = i * (L * UNROLL) + u * L
                xv = xb[pl.ds(o, L)]
                uv = xv * 32.0
                si = uv.astype(jnp.int32)
                si = jnp.minimum(jnp.maximum(si, 0), 32)
                xr = uv - si.astype(jnp.float32)
                y0 = plsc.load_gather(gt, [si])
                dd = plsc.load_gather(dt, [si])
                yb[pl.ds(o, L)] = y0 + dd * xr
            return carry

        lax.fori_loop(0, T // (L * UNROLL), vec, 0, unroll=False)

    def pair(p, carry):
        for b in range(2):
            c = 2 * p + b
            in_copy(c, b).wait()

            @pl.when(c >= 2)
            def _():
                out_copy(c - 2, b).wait()

            compute_chunk(b)
            out_copy(c, b).start()

            @pl.when(c + 2 < NCH)
            def _():
                in_copy(c + 2, b).start()
        return carry

    lax.fori_loop(0, NCH // 2, pair, 0)

    out_copy(NCH - 2, 0).wait()
    out_copy(NCH - 1, 1).wait()


@jax.jit
def kernel(x, g_param):
    g48 = jnp.zeros((_TBL,), jnp.float32).at[:33].set(g_param)
    mesh = plsc.VectorSubcoreMesh(
        core_axis_name="c", subcore_axis_name="s",
        num_cores=NC, num_subcores=NS)
    run = pl.kernel(
        _sc_body,
        out_type=jax.ShapeDtypeStruct((N_TOTAL,), jnp.float32),
        mesh=mesh,
        compiler_params=pltpu.CompilerParams(needs_layout_passes=False),
        scratch_types=[
            pltpu.VMEM((T,), jnp.float32),
            pltpu.VMEM((T,), jnp.float32),
            pltpu.VMEM((T,), jnp.float32),
            pltpu.VMEM((T,), jnp.float32),
            pltpu.VMEM((_TBL,), jnp.float32),
            pltpu.VMEM((_TBL,), jnp.float32),
            pltpu.SemaphoreType.DMA,
            pltpu.SemaphoreType.DMA,
            pltpu.SemaphoreType.DMA,
            pltpu.SemaphoreType.DMA,
        ],
    )
    y = run(x.reshape(N_TOTAL), g48)
    return y.reshape(N_ROWS, WIDTH)


# speedup vs baseline: 1.7004x; 1.7004x over previous
import functools

import jax
import jax.numpy as jnp
from jax import lax
from jax.experimental import pallas as pl
from jax.experimental.pallas import tpu as pltpu
from jax.experimental.pallas import tpu_sc as plsc

NC = 2
NS = 16
L = 16
NW = NC * NS

N_ROWS = 16384
WIDTH = 1920
N_TOTAL = N_ROWS * WIDTH
PER_W = N_TOTAL // NW
T = 24576
NCH = PER_W // T
UNROLL = 8

assert PER_W * NW == N_TOTAL and NCH * T == PER_W and T % (L * UNROLL) == 0

_TBL = 48


def _sc_body(x_hbm, g_hbm, out_hbm, xbuf0, xbuf1, ybuf0, ybuf1, gt, dt,
             sin0, sin1, sout0, sout1):
    wid = lax.axis_index("c") * NS + lax.axis_index("s")
    base = wid * PER_W
    xbufs = (xbuf0, xbuf1)
    ybufs = (ybuf0, ybuf1)
    sins = (sin0, sin1)
    souts = (sout0, sout1)

    pltpu.sync_copy(g_hbm, gt)
    for k in range(_TBL // L):
        i0 = lax.iota(jnp.int32, L) + (16 * k)
        i0 = jnp.minimum(i0, 32)
        i1 = jnp.minimum(i0 + 1, 32)
        dt[pl.ds(16 * k, L)] = (plsc.load_gather(gt, [i1])
                                - plsc.load_gather(gt, [i0]))

    def in_copy(c, b):
        return pltpu.make_async_copy(
            x_hbm.at[pl.ds(base + c * T, T)], xbufs[b], sins[b])

    def out_copy(c, b):
        return pltpu.make_async_copy(
            ybufs[b], out_hbm.at[pl.ds(base + c * T, T)], souts[b])

    in_copy(0, 0).start()
    in_copy(1, 1).start()

    def compute_chunk(b):
        xb = xbufs[b]
        yb = ybufs[b]

        @plsc.parallel_loop(0, T, step=L, unroll=UNROLL)
        def _(o):
            xv = xb[pl.ds(o, L)]
            uv = xv * 32.0
            si = uv.astype(jnp.int32)
            si = jnp.minimum(jnp.maximum(si, 0), 32)
            xr = uv - si.astype(jnp.float32)
            y0 = plsc.load_gather(gt, [si])
            dd = plsc.load_gather(dt, [si])
            yb[pl.ds(o, L)] = y0 + dd * xr

    def pair(p, carry):
        for b in range(2):
            c = 2 * p + b
            in_copy(c, b).wait()

            @pl.when(c >= 2)
            def _():
                out_copy(c - 2, b).wait()

            compute_chunk(b)
            out_copy(c, b).start()

            @pl.when(c + 2 < NCH)
            def _():
                in_copy(c + 2, b).start()
        return carry

    lax.fori_loop(0, NCH // 2, pair, 0)

    out_copy(NCH - 2, 0).wait()
    out_copy(NCH - 1, 1).wait()


@jax.jit
def kernel(x, g_param):
    g48 = jnp.zeros((_TBL,), jnp.float32).at[:33].set(g_param)
    mesh = plsc.VectorSubcoreMesh(
        core_axis_name="c", subcore_axis_name="s",
        num_cores=NC, num_subcores=NS)
    run = pl.kernel(
        _sc_body,
        out_type=jax.ShapeDtypeStruct((N_TOTAL,), jnp.float32),
        mesh=mesh,
        compiler_params=pltpu.CompilerParams(needs_layout_passes=False),
        scratch_types=[
            pltpu.VMEM((T,), jnp.float32),
            pltpu.VMEM((T,), jnp.float32),
            pltpu.VMEM((T,), jnp.float32),
            pltpu.VMEM((T,), jnp.float32),
            pltpu.VMEM((_TBL,), jnp.float32),
            pltpu.VMEM((_TBL,), jnp.float32),
            pltpu.SemaphoreType.DMA,
            pltpu.SemaphoreType.DMA,
            pltpu.SemaphoreType.DMA,
            pltpu.SemaphoreType.DMA,
        ],
    )
    y = run(x.reshape(N_TOTAL), g48)
    return y.reshape(N_ROWS, WIDTH)


# speedup vs baseline: 4.0370x; 2.3742x over previous
import functools

import jax
import jax.numpy as jnp
from jax import lax
from jax.experimental import pallas as pl
from jax.experimental.pallas import tpu as pltpu
from jax.experimental.pallas import tpu_sc as plsc

NC = 2
NS = 16
L = 16
NW = NC * NS

N_ROWS = 16384
WIDTH = 1920
ROWS_W = N_ROWS // NW
CH = 16
NCH = ROWS_W // CH
UNROLL = 8

_TBL = 48


def _sc_body(x_hbm, g_hbm, out_hbm, xbuf0, xbuf1, ybuf0, ybuf1, gt, dt,
             sin0, sin1, sout0, sout1):
    wid = lax.axis_index("c") * NS + lax.axis_index("s")
    base = wid * ROWS_W
    xbufs = (xbuf0, xbuf1)
    ybufs = (ybuf0, ybuf1)
    sins = (sin0, sin1)
    souts = (sout0, sout1)

    pltpu.sync_copy(g_hbm, gt)
    for k in range(_TBL // L):
        i0 = lax.iota(jnp.int32, L) + (16 * k)
        i0 = jnp.minimum(i0, 32)
        i1 = jnp.minimum(i0 + 1, 32)
        dt[pl.ds(16 * k, L)] = (plsc.load_gather(gt, [i1])
                                - plsc.load_gather(gt, [i0]))

    def in_copy(c, b):
        return pltpu.make_async_copy(
            x_hbm.at[pl.ds(base + c * CH, CH), :], xbufs[b], sins[b])

    def out_copy(c, b):
        return pltpu.make_async_copy(
            ybufs[b], out_hbm.at[pl.ds(base + c * CH, CH), :], souts[b])

    in_copy(0, 0).start()
    in_copy(1, 1).start()

    def compute_chunk(b):
        xb = xbufs[b]
        yb = ybufs[b]

        def row(r, carry):
            @plsc.parallel_loop(0, WIDTH, step=L, unroll=UNROLL)
            def _(o):
                xv = xb[r, pl.ds(o, L)]
                uv = xv * 32.0
                si = uv.astype(jnp.int32)
                si = jnp.minimum(jnp.maximum(si, 0), 32)
                xr = uv - si.astype(jnp.float32)
                y0 = plsc.load_gather(gt, [si])
                dd = plsc.load_gather(dt, [si])
                yb[r, pl.ds(o, L)] = y0 + dd * xr
            return carry

        lax.fori_loop(0, CH, row, 0)

    def pair(p, carry):
        for b in range(2):
            c = 2 * p + b
            in_copy(c, b).wait()

            @pl.when(c >= 2)
            def _():
                out_copy(c - 2, b).wait()

            compute_chunk(b)
            out_copy(c, b).start()

            @pl.when(c + 2 < NCH)
            def _():
                in_copy(c + 2, b).start()
        return carry

    lax.fori_loop(0, NCH // 2, pair, 0)

    out_copy(NCH - 2, 0).wait()
    out_copy(NCH - 1, 1).wait()


@jax.jit
def kernel(x, g_param):
    g48 = jnp.zeros((_TBL,), jnp.float32).at[:33].set(g_param)
    mesh = plsc.VectorSubcoreMesh(
        core_axis_name="c", subcore_axis_name="s",
        num_cores=NC, num_subcores=NS)
    run = pl.kernel(
        _sc_body,
        out_type=jax.ShapeDtypeStruct((N_ROWS, WIDTH), jnp.float32),
        mesh=mesh,
        compiler_params=pltpu.CompilerParams(needs_layout_passes=False),
        scratch_types=[
            pltpu.VMEM((CH, WIDTH), jnp.float32),
            pltpu.VMEM((CH, WIDTH), jnp.float32),
            pltpu.VMEM((CH, WIDTH), jnp.float32),
            pltpu.VMEM((CH, WIDTH), jnp.float32),
            pltpu.VMEM((_TBL,), jnp.float32),
            pltpu.VMEM((_TBL,), jnp.float32),
            pltpu.SemaphoreType.DMA,
            pltpu.SemaphoreType.DMA,
            pltpu.SemaphoreType.DMA,
            pltpu.SemaphoreType.DMA,
        ],
    )
    return run(x, g48)


# speedup vs baseline: 4.1768x; 1.0346x over previous
import functools

import jax
import jax.numpy as jnp
from jax import lax
from jax.experimental import pallas as pl
from jax.experimental.pallas import tpu as pltpu
from jax.experimental.pallas import tpu_sc as plsc

NC = 2
NS = 16
L = 16
NW = NC * NS

N_ROWS = 16384
WIDTH = 1920
ROWS_W = N_ROWS // NW
CH = 16
NCH = ROWS_W // CH
UNROLL = 8

_TBL = 48


def _sc_body(x_hbm, g_hbm, out_hbm, xbuf0, xbuf1, ybuf0, ybuf1, gt, dt,
             sin0, sin1, sout0, sout1):
    wid = lax.axis_index("c") * NS + lax.axis_index("s")
    base = wid * ROWS_W
    xbufs = (xbuf0, xbuf1)
    ybufs = (ybuf0, ybuf1)
    sins = (sin0, sin1)
    souts = (sout0, sout1)

    pltpu.sync_copy(g_hbm, gt)
    for k in range(_TBL // L):
        i0 = lax.iota(jnp.int32, L) + (16 * k)
        i0 = jnp.minimum(i0, 32)
        i1 = jnp.minimum(i0 + 1, 32)
        dt[pl.ds(16 * k, L)] = (plsc.load_gather(gt, [i1])
                                - plsc.load_gather(gt, [i0]))

    def in_copy(c, b):
        return pltpu.make_async_copy(
            x_hbm.at[pl.ds(base + c * CH, CH), :], xbufs[b], sins[b])

    def out_copy(c, b):
        return pltpu.make_async_copy(
            ybufs[b], out_hbm.at[pl.ds(base + c * CH, CH), :], souts[b])

    in_copy(0, 0).start()
    in_copy(1, 1).start()

    def compute_chunk(b):
        xb = xbufs[b]
        yb = ybufs[b]

        @plsc.parallel_loop(0, CH * WIDTH, step=L, unroll=UNROLL)
        def _(o):
            r = o // WIDTH
            c = o - r * WIDTH
            xv = xb[r, pl.ds(c, L)]
            uv = xv * 32.0
            si = uv.astype(jnp.int32)
            si = jnp.minimum(jnp.maximum(si, 0), 32)
            xr = uv - si.astype(jnp.float32)
            y0 = plsc.load_gather(gt, [si])
            dd = plsc.load_gather(dt, [si])
            yb[r, pl.ds(c, L)] = y0 + dd * xr

    def pair(p, carry):
        for b in range(2):
            c = 2 * p + b
            in_copy(c, b).wait()

            @pl.when(c >= 2)
            def _():
                out_copy(c - 2, b).wait()

            compute_chunk(b)
            out_copy(c, b).start()

            @pl.when(c + 2 < NCH)
            def _():
                in_copy(c + 2, b).start()
        return carry

    lax.fori_loop(0, NCH // 2, pair, 0)

    out_copy(NCH - 2, 0).wait()
    out_copy(NCH - 1, 1).wait()


@jax.jit
def kernel(x, g_param):
    g48 = jnp.zeros((_TBL,), jnp.float32).at[:33].set(g_param)
    mesh = plsc.VectorSubcoreMesh(
        core_axis_name="c", subcore_axis_name="s",
        num_cores=NC, num_subcores=NS)
    run = pl.kernel(
        _sc_body,
        out_type=jax.ShapeDtypeStruct((N_ROWS, WIDTH), jnp.float32),
        mesh=mesh,
        compiler_params=pltpu.CompilerParams(needs_layout_passes=False),
        scratch_types=[
            pltpu.VMEM((CH, WIDTH), jnp.float32),
            pltpu.VMEM((CH, WIDTH), jnp.float32),
            pltpu.VMEM((CH, WIDTH), jnp.float32),
            pltpu.VMEM((CH, WIDTH), jnp.float32),
            pltpu.VMEM((_TBL,), jnp.float32),
            pltpu.VMEM((_TBL,), jnp.float32),
            pltpu.SemaphoreType.DMA,
            pltpu.SemaphoreType.DMA,
            pltpu.SemaphoreType.DMA,
            pltpu.SemaphoreType.DMA,
        ],
    )
    return run(x, g48)
